# baseline (speedup 1.0000x reference)
import functools

import jax
import jax.numpy as jnp
from jax import lax
from jax.experimental import pallas as pl
from jax.experimental.pallas import tpu as pltpu
from jax.experimental.pallas import tpu_sc as plsc

B, K, N, H, D, KC = 16, 20, 1024, 4, 128, 16
IN_FEATS = 128
Nt = B * N
Kt = B * K
KP = 32
PB = 8
BIG = 3.0e38
PER_W = Kt // 32
f32 = jnp.float32


def _body(h_rec_ref, h0_ref, xr_ref, x0_ref,
          w_src_ref, pos_ref, d2o_ref, dx2o_ref):
    hi = jax.lax.Precision.HIGHEST
    w_src = w_src_ref[...]
    G = range(PB)

    zpad = jnp.zeros((N, 5), jnp.float32)
    srow_n = jax.lax.broadcasted_iota(jnp.int32, (8, N), 0)
    hb = [h_rec_ref[pl.ds(i * N, N), :] for i in G]
    x_rec_t = [jnp.where(srow_n == 3, 1.0, jnp.transpose(jnp.concatenate(
        [xr_ref[pl.ds(i * N, N), :], zpad], axis=1))) for i in G]
    x0_t = [jnp.transpose(jnp.concatenate(
        [x0_ref[pl.ds(i * N, N), :], zpad], axis=1)) for i in G]

    ft_src = [jnp.dot(hb[i], w_src, preferred_element_type=jnp.float32)
              for i in G]
    ft_dst = [jnp.dot(h0_ref[i], w_src, preferred_element_type=jnp.float32)
              for i in G]
    row = jax.lax.broadcasted_iota(jnp.int32, (H * D, KP), 0)
    bd = []
    for i in G:
        ft_dstT = jnp.transpose(ft_dst[i])
        bd.append(jnp.concatenate(
            [jnp.where((row >= h * D) & (row < (h + 1) * D), ft_dstT, 0.0)
             for h in range(H)], axis=1))
    e = [jnp.exp(jnp.dot(ft_src[i], bd[i],
                         preferred_element_type=jnp.float32)
                 / jnp.sqrt(jnp.float32(D))) for i in G]

    num = [jnp.dot(x_rec_t[i], e[i], preferred_element_type=jnp.float32,
                   precision=hi) for i in G]
    srow = jax.lax.broadcasted_iota(jnp.int32, (8, KP), 0)
    kp_pos_t, kp_pos = [], []
    for i in G:
        acc = jnp.zeros((8, KP), jnp.float32)
        for h in range(H):
            blk = num[i][:, h * KP:(h + 1) * KP]
            acc = acc + blk * (1.0 / blk[3:4, :])
        pt = jnp.where(srow < 3, acc * (1.0 / H), 0.0)
        kp_pos_t.append(pt)
        kp_pos.append(jnp.transpose(pt))
        pos_ref[i] = kp_pos[i]

    d2, dx2 = [], []
    for i in G:
        kpsq = jnp.sum(kp_pos[i] * kp_pos[i], axis=1, keepdims=True)
        x0sq = jnp.sum(x0_t[i] * x0_t[i], axis=0, keepdims=True)
        cross = jax.lax.dot_general(kp_pos_t[i], x0_t[i],
                                    (((0,), (0,)), ((), ())),
                                    preferred_element_type=jnp.float32)
        d2.append(kpsq + x0sq - 2.0 * cross)
        a = jnp.zeros((KP, N), jnp.float32)
        for c in range(3):
            dc = x_rec_t[i][c:c + 1, :] - kp_pos[i][:, c:c + 1]
            a = a + dc * dc
        dx2.append(a)

    for i in G:
        d2o_ref[i] = d2[i]
        dx2o_ref[i] = dx2[i]


@functools.partial(
    pl.kernel,
    mesh=plsc.VectorSubcoreMesh(core_axis_name="c", subcore_axis_name="s"),
    compiler_params=pltpu.CompilerParams(needs_layout_passes=False),
    out_type=[
        jax.ShapeDtypeStruct((Kt, 128), f32),
        jax.ShapeDtypeStruct((Kt, KC), f32),
    ],
    scratch_types=[
        pltpu.VMEM((N,), f32),
        pltpu.VMEM((N,), f32),
        pltpu.VMEM((KC, 128), f32),
        pltpu.VMEM((128,), f32),
        pltpu.VMEM((16,), f32),
        pltpu.SemaphoreType.DMA,
    ],
)
def sc_knn(d2_hbm, dx2_hbm, hr_hbm, hm_hbm, d2o_hbm,
           d2row_v, dx2row_v, hbuf_v, hm_v, d16_v, sem1):
    wid = lax.axis_index("s") * 2 + lax.axis_index("c")
    for t in range(PER_W):
        g = wid * PER_W + t
        b = g // K
        r = g % K
        row = b * KP + r
        pltpu.sync_copy(d2_hbm.at[row], d2row_v)
        pltpu.sync_copy(dx2_hbm.at[row], dx2row_v)

        def chunk_body(c, carry):
            keys, vals = carry
            ch = d2row_v[pl.ds(c * 16, 16)]
            cidx = c * 16 + lax.iota(jnp.int32, 16)
            sk, sv = plsc.sort_key_val(ch, cidx)
            rk = lax.rev(sk, (0,))
            rv = lax.rev(sv, (0,))
            m = keys <= rk
            nk = jnp.where(m, keys, rk)
            nv = jnp.where(m, vals, rv)
            out = plsc.sort_key_val(nk, nv)
            return (out[0], out[1])

        keys, vals = lax.fori_loop(
            0, N // 16, chunk_body,
            (jnp.full((16,), BIG, f32), jnp.zeros((16,), jnp.int32)))

        d16_v[...] = plsc.load_gather(dx2row_v, [vals])
        pltpu.sync_copy(d16_v, d2o_hbm.at[g])

        gidx = b * N + vals
        pltpu.async_copy(hr_hbm.at[gidx], hbuf_v, sem1).wait()
        for c8 in range(8):
            s = jnp.zeros((16,), f32)
            for rr in range(KC):
                s = s + hbuf_v[rr, pl.ds(c8 * 16, 16)]
            hm_v[pl.ds(c8 * 16, 16)] = s * (1.0 / KC)
        pltpu.sync_copy(hm_v, hm_hbm.at[g])


@functools.partial(jax.jit, static_argnames=("interpret",))
def _run(h_rec, h0_kp, x_rec, x0_rec, W_src, W_mlp, b_mlp, interpret=False):
    f32 = jnp.float32
    h0_pad = jnp.pad(h0_kp.reshape(B, K, IN_FEATS),
                     ((0, 0), (0, KP - K), (0, 0)))
    b2 = b_mlp.reshape(1, D)

    pos, d2m, dx2m = pl.pallas_call(
        _body,
        grid=(B // PB,),
        in_specs=[
            pl.BlockSpec((PB * N, IN_FEATS), lambda b: (b, 0)),
            pl.BlockSpec((PB, KP, IN_FEATS), lambda b: (b, 0, 0)),
            pl.BlockSpec((PB * N, 3), lambda b: (b, 0)),
            pl.BlockSpec((PB * N, 3), lambda b: (b, 0)),
            pl.BlockSpec((IN_FEATS, H * D), lambda b: (0, 0)),
        ],
        out_specs=[
            pl.BlockSpec((PB, KP, 8), lambda b: (b, 0, 0)),
            pl.BlockSpec((PB, KP, N), lambda b: (b, 0, 0)),
            pl.BlockSpec((PB, KP, N), lambda b: (b, 0, 0)),
        ],
        out_shape=[
            jax.ShapeDtypeStruct((B, KP, 8), f32),
            jax.ShapeDtypeStruct((B, KP, N), f32),
            jax.ShapeDtypeStruct((B, KP, N), f32),
        ],
        interpret=interpret,
    )(h_rec, h0_pad, x_rec, x0_rec, W_src)

    hm, d2sel = sc_knn(d2m.reshape(B * KP, N), dx2m.reshape(B * KP, N), h_rec)

    def _mlp_body(hm_ref, dsel_ref, wmlp_ref, bias_ref, out_ref):
        cat = jnp.concatenate([hm_ref[...], jnp.sqrt(dsel_ref[...])], axis=1)
        pre = (jnp.dot(cat, wmlp_ref[...], preferred_element_type=jnp.float32)
               + bias_ref[...])
        out_ref[...] = pre * jax.lax.logistic(pre)

    kp_feat = pl.pallas_call(
        _mlp_body,
        out_shape=jax.ShapeDtypeStruct((Kt, D), f32),
        interpret=interpret,
    )(hm, d2sel, W_mlp, b2)

    kp_pos = pos[:, :K, :3].reshape(Kt, 3)
    return kp_pos, kp_feat


def kernel(h_rec, h0_kp, x_rec, x0_rec, W_src, W_mlp, b_mlp,
           kp_batch_idx, edge_src, edge_dst):
    return _run(h_rec, h0_kp, x_rec, x0_rec, W_src, W_mlp, b_mlp)


# speedup vs baseline: 1.0481x; 1.0481x over previous
import functools

import jax
import jax.numpy as jnp
from jax import lax
from jax.experimental import pallas as pl
from jax.experimental.pallas import tpu as pltpu
from jax.experimental.pallas import tpu_sc as plsc

B, K, N, H, D, KC = 16, 20, 1024, 4, 128, 16
IN_FEATS = 128
Nt = B * N
Kt = B * K
KP = 32
PB = 8
BIG = 3.0e38
PER_W = Kt // 32
f32 = jnp.float32


def _body(h_rec_ref, h0_ref, xr_ref, x0_ref,
          w_src_ref, pos_ref, d2o_ref, dx2o_ref):
    hi = jax.lax.Precision.HIGHEST
    w_src = w_src_ref[...]
    G = range(PB)

    zpad = jnp.zeros((N, 5), jnp.float32)
    srow_n = jax.lax.broadcasted_iota(jnp.int32, (8, N), 0)
    hb = [h_rec_ref[pl.ds(i * N, N), :] for i in G]
    x_rec_t = [jnp.where(srow_n == 3, 1.0, jnp.transpose(jnp.concatenate(
        [xr_ref[pl.ds(i * N, N), :], zpad], axis=1))) for i in G]
    x0_t = [jnp.transpose(jnp.concatenate(
        [x0_ref[pl.ds(i * N, N), :], zpad], axis=1)) for i in G]

    ft_src = [jnp.dot(hb[i], w_src, preferred_element_type=jnp.float32)
              for i in G]
    ft_dst = [jnp.dot(h0_ref[i], w_src, preferred_element_type=jnp.float32)
              for i in G]
    row = jax.lax.broadcasted_iota(jnp.int32, (H * D, KP), 0)
    bd = []
    for i in G:
        ft_dstT = jnp.transpose(ft_dst[i])
        bd.append(jnp.concatenate(
            [jnp.where((row >= h * D) & (row < (h + 1) * D), ft_dstT, 0.0)
             for h in range(H)], axis=1))
    e = [jnp.exp(jnp.dot(ft_src[i], bd[i],
                         preferred_element_type=jnp.float32)
                 / jnp.sqrt(jnp.float32(D))) for i in G]

    num = [jnp.dot(x_rec_t[i], e[i], preferred_element_type=jnp.float32,
                   precision=hi) for i in G]
    srow = jax.lax.broadcasted_iota(jnp.int32, (8, KP), 0)
    kp_pos_t, kp_pos = [], []
    for i in G:
        acc = jnp.zeros((8, KP), jnp.float32)
        for h in range(H):
            blk = num[i][:, h * KP:(h + 1) * KP]
            acc = acc + blk * (1.0 / blk[3:4, :])
        pt = jnp.where(srow < 3, acc * (1.0 / H), 0.0)
        kp_pos_t.append(pt)
        kp_pos.append(jnp.transpose(pt))
        pos_ref[i] = kp_pos[i]

    d2, dx2 = [], []
    for i in G:
        kpsq = jnp.sum(kp_pos[i] * kp_pos[i], axis=1, keepdims=True)
        x0sq = jnp.sum(x0_t[i] * x0_t[i], axis=0, keepdims=True)
        cross = jax.lax.dot_general(kp_pos_t[i], x0_t[i],
                                    (((0,), (0,)), ((), ())),
                                    preferred_element_type=jnp.float32)
        d2.append(kpsq + x0sq - 2.0 * cross)
        a = jnp.zeros((KP, N), jnp.float32)
        for c in range(3):
            dc = x_rec_t[i][c:c + 1, :] - kp_pos[i][:, c:c + 1]
            a = a + dc * dc
        dx2.append(a)

    for i in G:
        d2o_ref[i] = d2[i]
        dx2o_ref[i] = dx2[i]


@functools.partial(
    pl.kernel,
    mesh=plsc.VectorSubcoreMesh(core_axis_name="c", subcore_axis_name="s"),
    compiler_params=pltpu.CompilerParams(needs_layout_passes=False),
    out_type=[
        jax.ShapeDtypeStruct((Kt, 128), f32),
        jax.ShapeDtypeStruct((Kt, KC), f32),
    ],
    scratch_types=[
        pltpu.VMEM((N,), f32),
        pltpu.VMEM((N,), f32),
        pltpu.VMEM((N,), f32),
        pltpu.VMEM((N,), f32),
        pltpu.VMEM((KC, 128), f32),
        pltpu.VMEM((128,), f32),
        pltpu.VMEM((16,), f32),
        pltpu.SemaphoreType.DMA,
    ],
)
def sc_knn(d2_hbm, dx2_hbm, hr_hbm, hm_hbm, d2o_hbm,
           d2row_a, d2row_b, dx2row_a, dx2row_b, hbuf_v, hm_v, d16_v, sem1):
    wid = lax.axis_index("s") * 2 + lax.axis_index("c")
    for tp in range(PER_W // 2):
        ga = wid * PER_W + 2 * tp
        gb = ga + 1
        rows = []
        for g, d2r, dx2r in ((ga, d2row_a, dx2row_a), (gb, d2row_b, dx2row_b)):
            b = g // K
            r = g % K
            pltpu.sync_copy(d2_hbm.at[b * KP + r], d2r)
            pltpu.sync_copy(dx2_hbm.at[b * KP + r], dx2r)
            rows.append((g, b, d2r, dx2r))

        def chunk_body(c, carry):
            ka, va, kb, vb = carry
            cidx = c * 16 + lax.iota(jnp.int32, 16)
            cha = d2row_a[pl.ds(c * 16, 16)]
            chb = d2row_b[pl.ds(c * 16, 16)]
            ska, sva = plsc.sort_key_val(cha, cidx)
            skb, svb = plsc.sort_key_val(chb, cidx)
            rka, rva = lax.rev(ska, (0,)), lax.rev(sva, (0,))
            rkb, rvb = lax.rev(skb, (0,)), lax.rev(svb, (0,))
            ma = ka <= rka
            mb = kb <= rkb
            oa = plsc.sort_key_val(jnp.where(ma, ka, rka), jnp.where(ma, va, rva))
            ob = plsc.sort_key_val(jnp.where(mb, kb, rkb), jnp.where(mb, vb, rvb))
            return (oa[0], oa[1], ob[0], ob[1])

        init = (jnp.full((16,), BIG, f32), jnp.zeros((16,), jnp.int32))
        ka, va, kb, vb = lax.fori_loop(0, N // 16, chunk_body, init + init)

        for (g, b, d2r, dx2r), vals in zip(rows, (va, vb)):
            d16_v[...] = plsc.load_gather(dx2r, [vals])
            pltpu.sync_copy(d16_v, d2o_hbm.at[g])
            gidx = b * N + vals
            pltpu.async_copy(hr_hbm.at[gidx], hbuf_v, sem1).wait()
            for c8 in range(8):
                sacc = jnp.zeros((16,), f32)
                for rr in range(KC):
                    sacc = sacc + hbuf_v[rr, pl.ds(c8 * 16, 16)]
                hm_v[pl.ds(c8 * 16, 16)] = sacc * (1.0 / KC)
            pltpu.sync_copy(hm_v, hm_hbm.at[g])


@functools.partial(jax.jit, static_argnames=("interpret",))
def _run(h_rec, h0_kp, x_rec, x0_rec, W_src, W_mlp, b_mlp, interpret=False):
    f32 = jnp.float32
    h0_pad = jnp.pad(h0_kp.reshape(B, K, IN_FEATS),
                     ((0, 0), (0, KP - K), (0, 0)))
    b2 = b_mlp.reshape(1, D)

    pos, d2m, dx2m = pl.pallas_call(
        _body,
        grid=(B // PB,),
        in_specs=[
            pl.BlockSpec((PB * N, IN_FEATS), lambda b: (b, 0)),
            pl.BlockSpec((PB, KP, IN_FEATS), lambda b: (b, 0, 0)),
            pl.BlockSpec((PB * N, 3), lambda b: (b, 0)),
            pl.BlockSpec((PB * N, 3), lambda b: (b, 0)),
            pl.BlockSpec((IN_FEATS, H * D), lambda b: (0, 0)),
        ],
        out_specs=[
            pl.BlockSpec((PB, KP, 8), lambda b: (b, 0, 0)),
            pl.BlockSpec((PB, KP, N), lambda b: (b, 0, 0)),
            pl.BlockSpec((PB, KP, N), lambda b: (b, 0, 0)),
        ],
        out_shape=[
            jax.ShapeDtypeStruct((B, KP, 8), f32),
            jax.ShapeDtypeStruct((B, KP, N), f32),
            jax.ShapeDtypeStruct((B, KP, N), f32),
        ],
        interpret=interpret,
    )(h_rec, h0_pad, x_rec, x0_rec, W_src)

    hm, d2sel = sc_knn(d2m.reshape(B * KP, N), dx2m.reshape(B * KP, N), h_rec)

    def _mlp_body(hm_ref, dsel_ref, wmlp_ref, bias_ref, out_ref):
        cat = jnp.concatenate([hm_ref[...], jnp.sqrt(dsel_ref[...])], axis=1)
        pre = (jnp.dot(cat, wmlp_ref[...], preferred_element_type=jnp.float32)
               + bias_ref[...])
        out_ref[...] = pre * jax.lax.logistic(pre)

    kp_feat = pl.pallas_call(
        _mlp_body,
        out_shape=jax.ShapeDtypeStruct((Kt, D), f32),
        interpret=interpret,
    )(hm, d2sel, W_mlp, b2)

    kp_pos = pos[:, :K, :3].reshape(Kt, 3)
    return kp_pos, kp_feat


def kernel(h_rec, h0_kp, x_rec, x0_rec, W_src, W_mlp, b_mlp,
           kp_batch_idx, edge_src, edge_dst):
    return _run(h_rec, h0_kp, x_rec, x0_rec, W_src, W_mlp, b_mlp)
